# baseline (speedup 1.0000x reference)
import jax
import jax.numpy as jnp
from jax import lax
from jax.experimental import pallas as pl
from jax.experimental.pallas import tpu as pltpu
from jax.experimental.pallas import tpu_sc as plsc

ROWS = 16384
COLS = 2048
LANES = 16
NUM_WORKERS = 32
ROWS_PER_W = ROWS // NUM_WORKERS
BLK_I = 16
BLK_O = 8
STEPS = ROWS_PER_W // BLK_I
NPAIR = STEPS // 2
CHUNKS = COLS // LANES


def _permute_body(x_hbm, perm_hbm, out_hbm, perm_v,
                  in_a, in_b, out_a, out_b,
                  isem_a, isem_b, osem_a, osem_b, psem):
    core = lax.axis_index("c")
    sub = lax.axis_index("s")
    wid = sub * 2 + core
    row0 = wid * ROWS_PER_W

    in_bufs = ((in_a, isem_a), (in_b, isem_b))
    out_bufs = ((out_a, osem_a), (out_b, osem_b))

    perm_cp = pltpu.make_async_copy(perm_hbm, perm_v, psem)
    perm_cp.start()
    for b in range(2):
        pltpu.make_async_copy(
            x_hbm.at[pl.ds(row0 + b * BLK_I, BLK_I)], in_bufs[b][0],
            in_bufs[b][1]).start()
    perm_cp.wait()

    row_ids = [jnp.full((LANES,), r, dtype=jnp.int32) for r in range(BLK_I)]

    def pair(p, carry):
        for b in range(2):
            in_v, isem = in_bufs[b]
            t = p * 2 + b
            base = row0 + t * BLK_I
            pltpu.make_async_copy(
                x_hbm.at[pl.ds(base, BLK_I)], in_v, isem).wait()

            for h in range(2):
                out_v, osem = out_bufs[h]
                obase = base + h * BLK_O

                @pl.when(t >= 1)
                def _drain():
                    pltpu.make_async_copy(
                        out_v, out_hbm.at[pl.ds(obase - BLK_I, BLK_O)], osem
                    ).wait()

                @plsc.parallel_loop(0, CHUNKS, unroll=16)
                def chunk(c):
                    col = c * LANES
                    idx = perm_v[pl.ds(col, LANES)]
                    for r in range(BLK_O):
                        vals = plsc.load_gather(
                            in_v, [row_ids[h * BLK_O + r], idx])
                        out_v[r, pl.ds(col, LANES)] = vals

                pltpu.make_async_copy(
                    out_v, out_hbm.at[pl.ds(obase, BLK_O)], osem).start()

            @pl.when(t + 2 < STEPS)
            def _prefetch():
                pltpu.make_async_copy(
                    x_hbm.at[pl.ds(base + 2 * BLK_I, BLK_I)], in_v, isem
                ).start()

        return carry

    lax.fori_loop(0, NPAIR, pair, 0)

    for h in range(2):
        obase = row0 + (STEPS - 1) * BLK_I + h * BLK_O
        pltpu.make_async_copy(
            out_bufs[h][0], out_hbm.at[pl.ds(obase, BLK_O)],
            out_bufs[h][1]).wait()


@jax.jit
def _permute(x, perm):
    mesh = plsc.VectorSubcoreMesh(core_axis_name="c", subcore_axis_name="s")
    run = pl.kernel(
        _permute_body,
        mesh=mesh,
        compiler_params=pltpu.CompilerParams(needs_layout_passes=False),
        out_type=jax.ShapeDtypeStruct((ROWS, COLS), jnp.float32),
        scratch_types=[
            pltpu.VMEM((COLS,), jnp.int32),
            pltpu.VMEM((BLK_I, COLS), jnp.float32),
            pltpu.VMEM((BLK_I, COLS), jnp.float32),
            pltpu.VMEM((BLK_O, COLS), jnp.float32),
            pltpu.VMEM((BLK_O, COLS), jnp.float32),
            pltpu.SemaphoreType.DMA,
            pltpu.SemaphoreType.DMA,
            pltpu.SemaphoreType.DMA,
            pltpu.SemaphoreType.DMA,
            pltpu.SemaphoreType.DMA,
        ],
    )
    return run(x, perm)


def kernel(x, perm, perm_inv):
    out = _permute(x, perm.astype(jnp.int32))
    return (out, 0)


# speedup vs baseline: 1.1120x; 1.1120x over previous
import jax
import jax.numpy as jnp
from jax import lax
from jax.experimental import pallas as pl
from jax.experimental.pallas import tpu as pltpu
from jax.experimental.pallas import tpu_sc as plsc

ROWS = 16384
COLS = 2048
LANES = 16
NUM_WORKERS = 32
ROWS_PER_W = ROWS // NUM_WORKERS
BLK_I = 16
BLK_O = 8
STEPS = ROWS_PER_W // BLK_I
NPAIR = STEPS // 2
CHUNKS = COLS // LANES


def _permute_body(x_hbm, perm_hbm, out_hbm, perm_v,
                  in_a, in_b, out_a, out_b,
                  isem_a, isem_b, osem_a, osem_b, psem):
    core = lax.axis_index("c")
    sub = lax.axis_index("s")
    wid = sub * 2 + core
    row0 = wid * ROWS_PER_W

    in_bufs = ((in_a, isem_a), (in_b, isem_b))
    out_bufs = ((out_a, osem_a), (out_b, osem_b))

    perm_cp = pltpu.make_async_copy(perm_hbm, perm_v, psem)
    perm_cp.start()
    for b in range(2):
        pltpu.make_async_copy(
            x_hbm.at[pl.ds(row0 + b * BLK_I, BLK_I)], in_bufs[b][0],
            in_bufs[b][1]).start()
    perm_cp.wait()

    row_ids = [jnp.full((LANES,), r, dtype=jnp.int32) for r in range(BLK_I)]

    def pair(p, carry):
        for b in range(2):
            in_v, isem = in_bufs[b]
            t = p * 2 + b
            base = row0 + t * BLK_I
            pltpu.make_async_copy(
                x_hbm.at[pl.ds(base, BLK_I)], in_v, isem).wait()

            for h in range(2):
                out_v, osem = out_bufs[h]
                obase = base + h * BLK_O

                @pl.when(t >= 1)
                def _drain():
                    pltpu.make_async_copy(
                        out_v, out_hbm.at[pl.ds(obase - BLK_I, BLK_O)], osem
                    ).wait()

                @plsc.parallel_loop(0, CHUNKS, unroll=8)
                def chunk(c):
                    col = c * LANES
                    idx = perm_v[pl.ds(col, LANES)]
                    for r in range(BLK_O):
                        vals = plsc.load_gather(
                            in_v, [row_ids[h * BLK_O + r], idx])
                        out_v[r, pl.ds(col, LANES)] = vals

                pltpu.make_async_copy(
                    out_v, out_hbm.at[pl.ds(obase, BLK_O)], osem).start()

            @pl.when(t + 2 < STEPS)
            def _prefetch():
                pltpu.make_async_copy(
                    x_hbm.at[pl.ds(base + 2 * BLK_I, BLK_I)], in_v, isem
                ).start()

        return carry

    lax.fori_loop(0, NPAIR, pair, 0)

    for h in range(2):
        obase = row0 + (STEPS - 1) * BLK_I + h * BLK_O
        pltpu.make_async_copy(
            out_bufs[h][0], out_hbm.at[pl.ds(obase, BLK_O)],
            out_bufs[h][1]).wait()


@jax.jit
def _permute(x, perm):
    mesh = plsc.VectorSubcoreMesh(core_axis_name="c", subcore_axis_name="s")
    run = pl.kernel(
        _permute_body,
        mesh=mesh,
        compiler_params=pltpu.CompilerParams(needs_layout_passes=False),
        out_type=jax.ShapeDtypeStruct((ROWS, COLS), jnp.float32),
        scratch_types=[
            pltpu.VMEM((COLS,), jnp.int32),
            pltpu.VMEM((BLK_I, COLS), jnp.float32),
            pltpu.VMEM((BLK_I, COLS), jnp.float32),
            pltpu.VMEM((BLK_O, COLS), jnp.float32),
            pltpu.VMEM((BLK_O, COLS), jnp.float32),
            pltpu.SemaphoreType.DMA,
            pltpu.SemaphoreType.DMA,
            pltpu.SemaphoreType.DMA,
            pltpu.SemaphoreType.DMA,
            pltpu.SemaphoreType.DMA,
        ],
    )
    return run(x, perm)


def kernel(x, perm, perm_inv):
    out = _permute(x, perm.astype(jnp.int32))
    return (out, 0)
